# Initial kernel scaffold; baseline (speedup 1.0000x reference)
import jax
import jax.numpy as jnp
from jax.experimental import pallas as pl


def kernel(x):
    raise NotImplementedError("write your pallas kernel here")


# speedup vs baseline: 6.7018x; 6.7018x over previous
import jax
import jax.numpy as jnp
from jax.experimental import pallas as pl
from jax.experimental.pallas import tpu as pltpu

_BLOCK_ROWS = 256


def _mxq84(x):
    ax = jnp.abs(x)
    e = jnp.ceil(jnp.log2(jnp.where(ax > 0, ax, 1.0)))
    e = jnp.clip(e, -8.0, 7.0)
    scale = jnp.exp2(e - 7.0)
    m = jnp.clip(jnp.round(x / scale), -128.0, 127.0)
    return m * scale


def _softmax_body(x_ref, o_ref):
    x = x_ref[...]
    qx = _mxq84(x)

    LOG2E = 1.4375
    new_mx = qx * LOG2E
    new_mx = jnp.clip(jnp.floor(new_mx * 128.0), -1024.0, 1023.0) / 128.0
    n = jnp.floor(new_mx)
    r = new_mx - n
    mexp = jnp.clip(jnp.round(jnp.exp2(r) * 64.0), -128.0, 127.0)

    M = n
    rows = M.shape[0]
    for k in (1, 2, 4, 8, 16, 32, 64, 128, 256, 512):
        prev = jnp.concatenate(
            [jnp.full((rows, k), -1000.0, M.dtype), M[:, :-k]], axis=1)
        M = jnp.maximum(M, prev)
    m_fin = jnp.max(n, axis=1, keepdims=True)

    macc = mexp * 16.0
    t = jnp.floor(macc * jnp.exp2(n - M))
    v = t * jnp.exp2(M - m_fin)
    th = jnp.floor(v)
    tl = (v - th) * 32768.0
    s_hi = jnp.sum(th, axis=1, keepdims=True)
    lo1 = jnp.sum(tl[:, :512], axis=1, keepdims=True)
    lo2 = jnp.sum(tl[:, 512:], axis=1, keepdims=True)
    s_lo = lo1.astype(jnp.int32) + lo2.astype(jnp.int32)
    mexp_sum = s_hi + (s_lo >> 15).astype(jnp.float32)

    mout = jnp.floor(mexp * 4096.0 / mexp_sum)
    qout = mout * jnp.exp2(n - m_fin) / 256.0
    o_ref[...] = _mxq84(_mxq84(qout))


def kernel(x):
    n_rows, n_feat = x.shape
    return pl.pallas_call(
        _softmax_body,
        grid=(n_rows // _BLOCK_ROWS,),
        in_specs=[pl.BlockSpec((_BLOCK_ROWS, n_feat), lambda i: (i, 0))],
        out_specs=pl.BlockSpec((_BLOCK_ROWS, n_feat), lambda i: (i, 0)),
        out_shape=jax.ShapeDtypeStruct((n_rows, n_feat), jnp.float32),
        compiler_params=pltpu.CompilerParams(
            dimension_semantics=("parallel",),
        ),
    )(x)
